# baseline (speedup 1.0000x reference)
import jax
import jax.numpy as jnp
from jax import lax
from jax.experimental import pallas as pl
from jax.experimental.pallas import tpu as pltpu
from jax.experimental.pallas import tpu_sc as plsc

N = 10000
E = 320000
F = 128
FH = F // 2
NGRAPH = 128
NCLS = 10

NC = 2
NS = 16
NW = NC * NS
EPW = E // NW
CHUNK = 80
NCHUNK = EPW // CHUNK
RPT = 624
ZBLK = 48
REM = N - NS * RPT

_mesh = plsc.VectorSubcoreMesh(core_axis_name="c", subcore_axis_name="s")


def _lane_bcast(vec, lane):
    idx = jnp.full((16, 1), lane, jnp.int32)
    return lax.gather(
        vec, idx,
        lax.GatherDimensionNumbers(
            offset_dims=(), collapsed_slice_dims=(0,), start_index_map=(0,)),
        slice_sizes=(1,),
        mode=lax.GatherScatterMode.PROMISE_IN_BOUNDS)


def _aggregate_body(h_hbm, src_hbm, dst_hbm, ew_hbm, out0_hbm, out1_hbm,
                    srcv0, srcv1, dstv0, dstv1, ewv0, ewv1,
                    rowsv0, rowsv1, obuf, accum,
                    sem_i0, sem_i1, sem_r0, sem_r1):
    c = lax.axis_index("c")
    s = lax.axis_index("s")
    w = s * NC + c

    def zrow(r, carry):
        for k in range(F // 16):
            obuf[r, pl.ds(k * 16, 16)] = jnp.zeros((16,), jnp.float32)
        return carry
    lax.fori_loop(0, ZBLK, zrow, 0)
    for k in range(RPT // ZBLK):
        pltpu.sync_copy(obuf, accum.at[pl.ds(s * RPT + k * ZBLK, ZBLK)])

    @pl.when(s == NS - 1)
    def _():
        pltpu.sync_copy(obuf.at[pl.ds(0, REM)], accum.at[pl.ds(NS * RPT, REM)])
    plsc.subcore_barrier()

    srcv = (srcv0, srcv1)
    dstv = (dstv0, dstv1)
    ewv = (ewv0, ewv1)
    rowsv = (rowsv0, rowsv1)
    sem_i = (sem_i0, sem_i1)
    sem_r = (sem_r0, sem_r1)

    def stage_idx(i, p):
        pltpu.async_copy(src_hbm.at[w, i], srcv[p], sem_i[p])
        pltpu.async_copy(dst_hbm.at[w, i], dstv[p], sem_i[p])
        pltpu.async_copy(ew_hbm.at[w, i], ewv[p], sem_i[p])

    def wait_idx(i, p):
        pltpu.make_async_copy(src_hbm.at[w, i], srcv[p], sem_i[p]).wait()
        pltpu.make_async_copy(dst_hbm.at[w, i], dstv[p], sem_i[p]).wait()
        pltpu.make_async_copy(ew_hbm.at[w, i], ewv[p], sem_i[p]).wait()

    def body(i, p):
        pltpu.make_async_copy(h_hbm.at[srcv[p].at[0]], rowsv[p],
                              sem_r[p]).wait()

        @pl.when(i + 1 < NCHUNK)
        def _():
            wait_idx(i + 1, 1 - p)
            pltpu.async_copy(h_hbm.at[srcv[1 - p].at[0]], rowsv[1 - p],
                             sem_r[1 - p])

        for j in range(CHUNK // 16):
            ew16 = ewv[p][0, pl.ds(j * 16, 16)]
            for l in range(16):
                w16 = _lane_bcast(ew16, l)
                e = j * 16 + l
                for k in range(F // 16):
                    sl = pl.ds(k * 16, 16)
                    rowsv[p][e, sl] = rowsv[p][e, sl] * w16

        pltpu.sync_copy(rowsv[p], accum.at[dstv[p].at[0]], add=True)

        @pl.when(i + 2 < NCHUNK)
        def _():
            stage_idx(i + 2, p)

    stage_idx(0, 0)
    stage_idx(1, 1)
    wait_idx(0, 0)
    pltpu.async_copy(h_hbm.at[srcv[0].at[0]], rowsv[0], sem_r[0])

    def pair(g, carry):
        body(g, 0)
        body(g + 1, 1)
        return carry
    lax.fori_loop(0, (NCHUNK - 1) // 2, lambda g, cc: pair(2 * g, cc), 0)
    body(NCHUNK - 1, 0)
    plsc.subcore_barrier()

    def writeout(out_ref):
        pltpu.sync_copy(accum.at[pl.ds(s * RPT, RPT)],
                        out_ref.at[pl.ds(s * RPT, RPT)])

        @pl.when(s == NS - 1)
        def _():
            pltpu.sync_copy(accum.at[pl.ds(NS * RPT, REM)],
                            out_ref.at[pl.ds(NS * RPT, REM)])

    @pl.when(c == 0)
    def _():
        writeout(out0_hbm)

    @pl.when(c == 1)
    def _():
        writeout(out1_hbm)


_aggregate = pl.kernel(
    _aggregate_body,
    out_type=(jax.ShapeDtypeStruct((N, F), jnp.float32),
              jax.ShapeDtypeStruct((N, F), jnp.float32)),
    mesh=_mesh,
    scratch_types=[
        pltpu.VMEM((1, CHUNK), jnp.int32),
        pltpu.VMEM((1, CHUNK), jnp.int32),
        pltpu.VMEM((1, CHUNK), jnp.int32),
        pltpu.VMEM((1, CHUNK), jnp.int32),
        pltpu.VMEM((1, CHUNK), jnp.float32),
        pltpu.VMEM((1, CHUNK), jnp.float32),
        pltpu.VMEM((CHUNK, F), jnp.float32),
        pltpu.VMEM((CHUNK, F), jnp.float32),
        pltpu.VMEM((ZBLK, F), jnp.float32),
        pltpu.VMEM_SHARED((N, F), jnp.float32),
        pltpu.SemaphoreType.DMA,
        pltpu.SemaphoreType.DMA,
        pltpu.SemaphoreType.DMA,
        pltpu.SemaphoreType.DMA,
    ],
)


BLK = 1000


def _mlp_body(h_ref, a0_ref, a1_ref, w1_ref, b1_ref, gb_ref,
              w2_ref, b2_ref, out_ref, stats_ref, z_ref):
    p = pl.program_id(0)
    b = pl.program_id(1)

    @pl.when(jnp.logical_and(p == 0, b == 0))
    def _():
        stats_ref[...] = jnp.zeros_like(stats_ref)

    @pl.when(p == 0)
    def _():
        acc = h_ref[...] + a0_ref[...] + a1_ref[...]
        z = jnp.dot(acc, w1_ref[...],
                    preferred_element_type=jnp.float32) + b1_ref[...]
        stats_ref[0:1, :] += jnp.sum(z, axis=0, keepdims=True)
        stats_ref[1:2, :] += jnp.sum(z * z, axis=0, keepdims=True)
        z_ref[pl.ds(b * BLK, BLK), :] = z

    @pl.when(p == 1)
    def _():
        z = z_ref[pl.ds(b * BLK, BLK), :]
        mean = stats_ref[0:1, :] * (1.0 / N)
        var = stats_ref[1:2, :] * (1.0 / N) - mean * mean
        zn = (z - mean) * lax.rsqrt(var + 1e-5) * gb_ref[0:1, :] + gb_ref[1:2, :]
        zr = jnp.maximum(zn, 0.0)
        out_ref[...] = jnp.maximum(
            jnp.dot(zr, w2_ref[...], preferred_element_type=jnp.float32)
            + b2_ref[...], 0.0)


def _mlp(h, a0, a1, p):
    gb = jnp.stack([p["gamma"], p["beta"]])
    return pl.pallas_call(
        _mlp_body,
        grid=(2, N // BLK),
        in_specs=[
            pl.BlockSpec((BLK, F), lambda gp, b: (b * (1 - gp), 0)),
            pl.BlockSpec((BLK, F), lambda gp, b: (b * (1 - gp), 0)),
            pl.BlockSpec((BLK, F), lambda gp, b: (b * (1 - gp), 0)),
            pl.BlockSpec((F, F), lambda gp, b: (0, 0)),
            pl.BlockSpec((1, F), lambda gp, b: (0, 0)),
            pl.BlockSpec((2, F), lambda gp, b: (0, 0)),
            pl.BlockSpec((F, F), lambda gp, b: (0, 0)),
            pl.BlockSpec((1, F), lambda gp, b: (0, 0)),
        ],
        out_specs=pl.BlockSpec((BLK, F), lambda gp, b: (b * gp, 0)),
        out_shape=jax.ShapeDtypeStruct((N, F), jnp.float32),
        scratch_shapes=[pltpu.VMEM((2, F), jnp.float32),
                        pltpu.VMEM((N, F), jnp.float32)],
    )(h, a0, a1, p["w1"], p["b1"][None, :], gb, p["w2"], p["b2"][None, :])


def _cls_body(h_ref, batch_ref, l1w_ref, l1b_ref, l2w_ref, l2b_ref,
              out_ref, acc_ref):
    b = pl.program_id(0)

    @pl.when(b == 0)
    def _():
        acc_ref[...] = jnp.zeros_like(acc_ref)

    oh = (batch_ref[...] == lax.broadcasted_iota(
        jnp.int32, (BLK, NGRAPH), 1)).astype(jnp.float32)
    acc_ref[...] += lax.dot_general(
        oh, h_ref[...], (((0,), (0,)), ((), ())),
        preferred_element_type=jnp.float32)

    @pl.when(b == N // BLK - 1)
    def _():
        g = acc_ref[...]
        t = jnp.maximum(
            jnp.dot(g, l1w_ref[...], preferred_element_type=jnp.float32)
            + l1b_ref[...], 0.0)
        logits = jnp.dot(t, l2w_ref[...], preferred_element_type=jnp.float32) \
            + l2b_ref[...]
        m = jnp.max(logits, axis=1, keepdims=True)
        lse = jnp.log(jnp.sum(jnp.exp(logits - m), axis=1, keepdims=True)) + m
        out_ref[...] = logits - lse


def _classify(h, batch, params):
    return pl.pallas_call(
        _cls_body,
        grid=(N // BLK,),
        in_specs=[
            pl.BlockSpec((BLK, F), lambda b: (b, 0)),
            pl.BlockSpec((BLK, 1), lambda b: (b, 0)),
            pl.BlockSpec((F, F), lambda b: (0, 0)),
            pl.BlockSpec((1, F), lambda b: (0, 0)),
            pl.BlockSpec((F, NCLS), lambda b: (0, 0)),
            pl.BlockSpec((1, NCLS), lambda b: (0, 0)),
        ],
        out_specs=pl.BlockSpec((NGRAPH, NCLS), lambda b: (0, 0)),
        out_shape=jax.ShapeDtypeStruct((NGRAPH, NCLS), jnp.float32),
        scratch_shapes=[pltpu.VMEM((NGRAPH, F), jnp.float32)],
    )(h, batch[:, None], params["lin1_w"], params["lin1_b"][None, :],
      params["lin2_w"], params["lin2_b"][None, :])


def kernel(x, edge_index, edge_weight, batch, params):
    src = edge_index[0].reshape(NW, NCHUNK, 1, CHUNK)
    dst = edge_index[1].reshape(NW, NCHUNK, 1, CHUNK)
    ew = edge_weight.reshape(NW, NCHUNK, 1, CHUNK)
    h = x
    for name in ("conv1", "conv2", "conv3"):
        a0, a1 = _aggregate(h, src, dst, ew)
        h = _mlp(h, a0, a1, params[name])
    return _classify(h, batch, params)


# speedup vs baseline: 1.0114x; 1.0114x over previous
import jax
import jax.numpy as jnp
from jax import lax
from jax.experimental import pallas as pl
from jax.experimental.pallas import tpu as pltpu
from jax.experimental.pallas import tpu_sc as plsc

N = 10000
E = 320000
F = 128
FH = F // 2
NGRAPH = 128
NCLS = 10

NC = 2
NS = 16
NW = NC * NS
EPW = E // NW
CHUNK = 80
NCHUNK = EPW // CHUNK
RPT = 624
ZBLK = 48
REM = N - NS * RPT

_mesh = plsc.VectorSubcoreMesh(core_axis_name="c", subcore_axis_name="s")


def _lane_bcast(vec, lane):
    idx = jnp.full((16, 1), lane, jnp.int32)
    return lax.gather(
        vec, idx,
        lax.GatherDimensionNumbers(
            offset_dims=(), collapsed_slice_dims=(0,), start_index_map=(0,)),
        slice_sizes=(1,),
        mode=lax.GatherScatterMode.PROMISE_IN_BOUNDS)


def _aggregate_body(h_hbm, src_hbm, dst_hbm, ew_hbm, out0_hbm, out1_hbm,
                    srcv0, srcv1, dstv0, dstv1, ewv0, ewv1,
                    rowsv0, rowsv1, obuf, accum,
                    sem_i0, sem_i1, sem_r0, sem_r1):
    c = lax.axis_index("c")
    s = lax.axis_index("s")
    w = s * NC + c

    def zrow(r, carry):
        for k in range(F // 16):
            obuf[r, pl.ds(k * 16, 16)] = jnp.zeros((16,), jnp.float32)
        return carry
    lax.fori_loop(0, ZBLK, zrow, 0)
    for k in range(RPT // ZBLK):
        pltpu.sync_copy(obuf, accum.at[pl.ds(s * RPT + k * ZBLK, ZBLK)])

    @pl.when(s == NS - 1)
    def _():
        pltpu.sync_copy(obuf.at[pl.ds(0, REM)], accum.at[pl.ds(NS * RPT, REM)])
    plsc.subcore_barrier()

    srcv = (srcv0, srcv1)
    dstv = (dstv0, dstv1)
    ewv = (ewv0, ewv1)
    rowsv = (rowsv0, rowsv1)
    sem_i = (sem_i0, sem_i1)
    sem_r = (sem_r0, sem_r1)

    def stage_idx(i, p):
        pltpu.async_copy(src_hbm.at[w, i], srcv[p], sem_i[p])
        pltpu.async_copy(dst_hbm.at[w, i], dstv[p], sem_i[p])
        pltpu.async_copy(ew_hbm.at[w, i], ewv[p], sem_i[p])

    def wait_idx(i, p):
        pltpu.make_async_copy(src_hbm.at[w, i], srcv[p], sem_i[p]).wait()
        pltpu.make_async_copy(dst_hbm.at[w, i], dstv[p], sem_i[p]).wait()
        pltpu.make_async_copy(ew_hbm.at[w, i], ewv[p], sem_i[p]).wait()

    def body(i, p):
        pltpu.make_async_copy(h_hbm.at[srcv[p].at[0]], rowsv[p],
                              sem_r[p]).wait()

        @pl.when(i + 1 < NCHUNK)
        def _():
            wait_idx(i + 1, 1 - p)
            pltpu.async_copy(h_hbm.at[srcv[1 - p].at[0]], rowsv[1 - p],
                             sem_r[1 - p])

        for j in range(CHUNK // 16):
            ew16 = ewv[p][0, pl.ds(j * 16, 16)]
            for l in range(16):
                w16 = _lane_bcast(ew16, l)
                e = j * 16 + l
                for k in range(F // 16):
                    sl = pl.ds(k * 16, 16)
                    rowsv[p][e, sl] = rowsv[p][e, sl] * w16

        pltpu.sync_copy(rowsv[p], accum.at[dstv[p].at[0]], add=True)

        @pl.when(i + 2 < NCHUNK)
        def _():
            stage_idx(i + 2, p)

    stage_idx(0, 0)
    stage_idx(1, 1)
    wait_idx(0, 0)
    pltpu.async_copy(h_hbm.at[srcv[0].at[0]], rowsv[0], sem_r[0])

    def pair(g, carry):
        body(g, 0)
        body(g + 1, 1)
        return carry
    lax.fori_loop(0, (NCHUNK - 1) // 2, lambda g, cc: pair(2 * g, cc), 0)
    body(NCHUNK - 1, 0)
    plsc.subcore_barrier()

    def writeout(out_ref):
        for k in range(RPT // ZBLK):
            base = s * RPT + k * ZBLK
            pltpu.sync_copy(accum.at[pl.ds(base, ZBLK)],
                            out_ref.at[pl.ds(base, ZBLK)])

        @pl.when(s == NS - 1)
        def _():
            pltpu.sync_copy(accum.at[pl.ds(NS * RPT, REM)],
                            out_ref.at[pl.ds(NS * RPT, REM)])

    @pl.when(c == 0)
    def _():
        writeout(out0_hbm)

    @pl.when(c == 1)
    def _():
        writeout(out1_hbm)


_aggregate = pl.kernel(
    _aggregate_body,
    out_type=(jax.ShapeDtypeStruct((N, F), jnp.float32),
              jax.ShapeDtypeStruct((N, F), jnp.float32)),
    mesh=_mesh,
    scratch_types=[
        pltpu.VMEM((1, CHUNK), jnp.int32),
        pltpu.VMEM((1, CHUNK), jnp.int32),
        pltpu.VMEM((1, CHUNK), jnp.int32),
        pltpu.VMEM((1, CHUNK), jnp.int32),
        pltpu.VMEM((1, CHUNK), jnp.float32),
        pltpu.VMEM((1, CHUNK), jnp.float32),
        pltpu.VMEM((CHUNK, F), jnp.float32),
        pltpu.VMEM((CHUNK, F), jnp.float32),
        pltpu.VMEM((ZBLK, F), jnp.float32),
        pltpu.VMEM_SHARED((N, F), jnp.float32),
        pltpu.SemaphoreType.DMA,
        pltpu.SemaphoreType.DMA,
        pltpu.SemaphoreType.DMA,
        pltpu.SemaphoreType.DMA,
    ],
)


BLK = 1000


def _mlp_body(h_ref, a0_ref, a1_ref, w1_ref, b1_ref, gb_ref,
              w2_ref, b2_ref, out_ref, stats_ref, z_ref):
    p = pl.program_id(0)
    b = pl.program_id(1)

    @pl.when(jnp.logical_and(p == 0, b == 0))
    def _():
        stats_ref[...] = jnp.zeros_like(stats_ref)

    @pl.when(p == 0)
    def _():
        acc = h_ref[...] + a0_ref[...] + a1_ref[...]
        z = jnp.dot(acc, w1_ref[...],
                    preferred_element_type=jnp.float32) + b1_ref[...]
        stats_ref[0:1, :] += jnp.sum(z, axis=0, keepdims=True)
        stats_ref[1:2, :] += jnp.sum(z * z, axis=0, keepdims=True)
        z_ref[pl.ds(b * BLK, BLK), :] = z

    @pl.when(p == 1)
    def _():
        z = z_ref[pl.ds(b * BLK, BLK), :]
        mean = stats_ref[0:1, :] * (1.0 / N)
        var = stats_ref[1:2, :] * (1.0 / N) - mean * mean
        zn = (z - mean) * lax.rsqrt(var + 1e-5) * gb_ref[0:1, :] + gb_ref[1:2, :]
        zr = jnp.maximum(zn, 0.0)
        out_ref[...] = jnp.maximum(
            jnp.dot(zr, w2_ref[...], preferred_element_type=jnp.float32)
            + b2_ref[...], 0.0)


def _mlp(h, a0, a1, p):
    gb = jnp.stack([p["gamma"], p["beta"]])
    return pl.pallas_call(
        _mlp_body,
        grid=(2, N // BLK),
        in_specs=[
            pl.BlockSpec((BLK, F), lambda gp, b: (b * (1 - gp), 0)),
            pl.BlockSpec((BLK, F), lambda gp, b: (b * (1 - gp), 0)),
            pl.BlockSpec((BLK, F), lambda gp, b: (b * (1 - gp), 0)),
            pl.BlockSpec((F, F), lambda gp, b: (0, 0)),
            pl.BlockSpec((1, F), lambda gp, b: (0, 0)),
            pl.BlockSpec((2, F), lambda gp, b: (0, 0)),
            pl.BlockSpec((F, F), lambda gp, b: (0, 0)),
            pl.BlockSpec((1, F), lambda gp, b: (0, 0)),
        ],
        out_specs=pl.BlockSpec((BLK, F), lambda gp, b: (b * gp, 0)),
        out_shape=jax.ShapeDtypeStruct((N, F), jnp.float32),
        scratch_shapes=[pltpu.VMEM((2, F), jnp.float32),
                        pltpu.VMEM((N, F), jnp.float32)],
    )(h, a0, a1, p["w1"], p["b1"][None, :], gb, p["w2"], p["b2"][None, :])


def _cls_body(h_ref, batch_ref, l1w_ref, l1b_ref, l2w_ref, l2b_ref,
              out_ref, acc_ref):
    b = pl.program_id(0)

    @pl.when(b == 0)
    def _():
        acc_ref[...] = jnp.zeros_like(acc_ref)

    oh = (batch_ref[...] == lax.broadcasted_iota(
        jnp.int32, (BLK, NGRAPH), 1)).astype(jnp.float32)
    acc_ref[...] += lax.dot_general(
        oh, h_ref[...], (((0,), (0,)), ((), ())),
        preferred_element_type=jnp.float32)

    @pl.when(b == N // BLK - 1)
    def _():
        g = acc_ref[...]
        t = jnp.maximum(
            jnp.dot(g, l1w_ref[...], preferred_element_type=jnp.float32)
            + l1b_ref[...], 0.0)
        logits = jnp.dot(t, l2w_ref[...], preferred_element_type=jnp.float32) \
            + l2b_ref[...]
        m = jnp.max(logits, axis=1, keepdims=True)
        lse = jnp.log(jnp.sum(jnp.exp(logits - m), axis=1, keepdims=True)) + m
        out_ref[...] = logits - lse


def _classify(h, batch, params):
    return pl.pallas_call(
        _cls_body,
        grid=(N // BLK,),
        in_specs=[
            pl.BlockSpec((BLK, F), lambda b: (b, 0)),
            pl.BlockSpec((BLK, 1), lambda b: (b, 0)),
            pl.BlockSpec((F, F), lambda b: (0, 0)),
            pl.BlockSpec((1, F), lambda b: (0, 0)),
            pl.BlockSpec((F, NCLS), lambda b: (0, 0)),
            pl.BlockSpec((1, NCLS), lambda b: (0, 0)),
        ],
        out_specs=pl.BlockSpec((NGRAPH, NCLS), lambda b: (0, 0)),
        out_shape=jax.ShapeDtypeStruct((NGRAPH, NCLS), jnp.float32),
        scratch_shapes=[pltpu.VMEM((NGRAPH, F), jnp.float32)],
    )(h, batch[:, None], params["lin1_w"], params["lin1_b"][None, :],
      params["lin2_w"], params["lin2_b"][None, :])


def kernel(x, edge_index, edge_weight, batch, params):
    src = edge_index[0].reshape(NW, NCHUNK, 1, CHUNK)
    dst = edge_index[1].reshape(NW, NCHUNK, 1, CHUNK)
    ew = edge_weight.reshape(NW, NCHUNK, 1, CHUNK)
    h = x
    for name in ("conv1", "conv2", "conv3"):
        a0, a1 = _aggregate(h, src, dst, ew)
        h = _mlp(h, a0, a1, params[name])
    return _classify(h, batch, params)


# speedup vs baseline: 1.0259x; 1.0143x over previous
import jax
import jax.numpy as jnp
from jax import lax
from jax.experimental import pallas as pl
from jax.experimental.pallas import tpu as pltpu
from jax.experimental.pallas import tpu_sc as plsc

N = 10000
E = 320000
F = 128
FH = F // 2
NGRAPH = 128
NCLS = 10

NC = 2
NS = 16
NW = NC * NS
EPW = E // NW
CHUNK = 80
NCHUNK = EPW // CHUNK
RPT = 624
ZBLK = 48
REM = N - NS * RPT

_mesh = plsc.VectorSubcoreMesh(core_axis_name="c", subcore_axis_name="s")


def _lane_bcast(vec, lane):
    idx = jnp.full((16, 1), lane, jnp.int32)
    return lax.gather(
        vec, idx,
        lax.GatherDimensionNumbers(
            offset_dims=(), collapsed_slice_dims=(0,), start_index_map=(0,)),
        slice_sizes=(1,),
        mode=lax.GatherScatterMode.PROMISE_IN_BOUNDS)


def _aggregate_body(h_hbm, src_hbm, dst_hbm, ew_hbm, out0_hbm, out1_hbm,
                    srcv0, srcv1, dstv0, dstv1, ewv0, ewv1,
                    rowsv0, rowsv1, obuf, accum,
                    sem_i0, sem_i1, sem_r0, sem_r1):
    c = lax.axis_index("c")
    s = lax.axis_index("s")
    w = s * NC + c

    def zrow(r, carry):
        for k in range(F // 16):
            obuf[r, pl.ds(k * 16, 16)] = jnp.zeros((16,), jnp.float32)
        return carry
    lax.fori_loop(0, ZBLK, zrow, 0)
    for k in range(RPT // ZBLK):
        pltpu.sync_copy(obuf, accum.at[pl.ds(s * RPT + k * ZBLK, ZBLK)])

    @pl.when(s == NS - 1)
    def _():
        pltpu.sync_copy(obuf.at[pl.ds(0, REM)], accum.at[pl.ds(NS * RPT, REM)])
    plsc.subcore_barrier()

    srcv = (srcv0, srcv1)
    dstv = (dstv0, dstv1)
    ewv = (ewv0, ewv1)
    rowsv = (rowsv0, rowsv1)
    sem_i = (sem_i0, sem_i1)
    sem_r = (sem_r0, sem_r1)

    def stage_idx(i, p):
        pltpu.async_copy(src_hbm.at[w, i], srcv[p], sem_i[p])
        pltpu.async_copy(dst_hbm.at[w, i], dstv[p], sem_i[p])
        pltpu.async_copy(ew_hbm.at[w, i], ewv[p], sem_i[p])

    def wait_idx(i, p):
        pltpu.make_async_copy(src_hbm.at[w, i], srcv[p], sem_i[p]).wait()
        pltpu.make_async_copy(dst_hbm.at[w, i], dstv[p], sem_i[p]).wait()
        pltpu.make_async_copy(ew_hbm.at[w, i], ewv[p], sem_i[p]).wait()

    def body(i, p):
        pltpu.make_async_copy(h_hbm.at[srcv[p].at[0]], rowsv[p],
                              sem_r[p]).wait()

        @pl.when(i + 1 < NCHUNK)
        def _():
            wait_idx(i + 1, 1 - p)
            pltpu.async_copy(h_hbm.at[srcv[1 - p].at[0]], rowsv[1 - p],
                             sem_r[1 - p])

        for j in range(CHUNK // 16):
            ew16 = ewv[p][0, pl.ds(j * 16, 16)]
            for l in range(16):
                w16 = _lane_bcast(ew16, l)
                e = j * 16 + l
                for k in range(F // 16):
                    sl = pl.ds(k * 16, 16)
                    rowsv[p][e, sl] = rowsv[p][e, sl] * w16

        pltpu.sync_copy(rowsv[p], accum.at[dstv[p].at[0]], add=True)

        @pl.when(i + 2 < NCHUNK)
        def _():
            stage_idx(i + 2, p)

    stage_idx(0, 0)
    stage_idx(1, 1)
    wait_idx(0, 0)
    pltpu.async_copy(h_hbm.at[srcv[0].at[0]], rowsv[0], sem_r[0])

    def pair(g, carry):
        body(g, 0)
        body(g + 1, 1)
        return carry
    lax.fori_loop(0, (NCHUNK - 1) // 2, lambda g, cc: pair(2 * g, cc), 0)
    body(NCHUNK - 1, 0)
    plsc.subcore_barrier()

    def writeout(out_ref):
        for k in range(RPT // ZBLK):
            base = s * RPT + k * ZBLK
            pltpu.sync_copy(accum.at[pl.ds(base, ZBLK)], obuf)
            pltpu.sync_copy(obuf, out_ref.at[pl.ds(base, ZBLK)])

        @pl.when(s == NS - 1)
        def _():
            pltpu.sync_copy(accum.at[pl.ds(NS * RPT, REM)],
                            obuf.at[pl.ds(0, REM)])
            pltpu.sync_copy(obuf.at[pl.ds(0, REM)],
                            out_ref.at[pl.ds(NS * RPT, REM)])

    @pl.when(c == 0)
    def _():
        writeout(out0_hbm)

    @pl.when(c == 1)
    def _():
        writeout(out1_hbm)


_aggregate = pl.kernel(
    _aggregate_body,
    out_type=(jax.ShapeDtypeStruct((N, F), jnp.float32),
              jax.ShapeDtypeStruct((N, F), jnp.float32)),
    mesh=_mesh,
    scratch_types=[
        pltpu.VMEM((1, CHUNK), jnp.int32),
        pltpu.VMEM((1, CHUNK), jnp.int32),
        pltpu.VMEM((1, CHUNK), jnp.int32),
        pltpu.VMEM((1, CHUNK), jnp.int32),
        pltpu.VMEM((1, CHUNK), jnp.float32),
        pltpu.VMEM((1, CHUNK), jnp.float32),
        pltpu.VMEM((CHUNK, F), jnp.float32),
        pltpu.VMEM((CHUNK, F), jnp.float32),
        pltpu.VMEM((ZBLK, F), jnp.float32),
        pltpu.VMEM_SHARED((N, F), jnp.float32),
        pltpu.SemaphoreType.DMA,
        pltpu.SemaphoreType.DMA,
        pltpu.SemaphoreType.DMA,
        pltpu.SemaphoreType.DMA,
    ],
)


BLK = 1000


def _mlp_body(h_ref, a0_ref, a1_ref, w1_ref, b1_ref, gb_ref,
              w2_ref, b2_ref, out_ref, stats_ref, z_ref):
    p = pl.program_id(0)
    b = pl.program_id(1)

    @pl.when(jnp.logical_and(p == 0, b == 0))
    def _():
        stats_ref[...] = jnp.zeros_like(stats_ref)

    @pl.when(p == 0)
    def _():
        acc = h_ref[...] + a0_ref[...] + a1_ref[...]
        z = jnp.dot(acc, w1_ref[...],
                    preferred_element_type=jnp.float32) + b1_ref[...]
        stats_ref[0:1, :] += jnp.sum(z, axis=0, keepdims=True)
        stats_ref[1:2, :] += jnp.sum(z * z, axis=0, keepdims=True)
        z_ref[pl.ds(b * BLK, BLK), :] = z

    @pl.when(p == 1)
    def _():
        z = z_ref[pl.ds(b * BLK, BLK), :]
        mean = stats_ref[0:1, :] * (1.0 / N)
        var = stats_ref[1:2, :] * (1.0 / N) - mean * mean
        zn = (z - mean) * lax.rsqrt(var + 1e-5) * gb_ref[0:1, :] + gb_ref[1:2, :]
        zr = jnp.maximum(zn, 0.0)
        out_ref[...] = jnp.maximum(
            jnp.dot(zr, w2_ref[...], preferred_element_type=jnp.float32)
            + b2_ref[...], 0.0)


def _mlp(h, a0, a1, p):
    gb = jnp.stack([p["gamma"], p["beta"]])
    return pl.pallas_call(
        _mlp_body,
        grid=(2, N // BLK),
        in_specs=[
            pl.BlockSpec((BLK, F), lambda gp, b: (b * (1 - gp), 0)),
            pl.BlockSpec((BLK, F), lambda gp, b: (b * (1 - gp), 0)),
            pl.BlockSpec((BLK, F), lambda gp, b: (b * (1 - gp), 0)),
            pl.BlockSpec((F, F), lambda gp, b: (0, 0)),
            pl.BlockSpec((1, F), lambda gp, b: (0, 0)),
            pl.BlockSpec((2, F), lambda gp, b: (0, 0)),
            pl.BlockSpec((F, F), lambda gp, b: (0, 0)),
            pl.BlockSpec((1, F), lambda gp, b: (0, 0)),
        ],
        out_specs=pl.BlockSpec((BLK, F), lambda gp, b: (b * gp, 0)),
        out_shape=jax.ShapeDtypeStruct((N, F), jnp.float32),
        scratch_shapes=[pltpu.VMEM((2, F), jnp.float32),
                        pltpu.VMEM((N, F), jnp.float32)],
    )(h, a0, a1, p["w1"], p["b1"][None, :], gb, p["w2"], p["b2"][None, :])


def _mlp_cls_body(h_ref, a0_ref, a1_ref, w1_ref, b1_ref, gb_ref,
                  w2_ref, b2_ref, batch_ref, l1w_ref, l1b_ref,
                  l2w_ref, l2b_ref, out_ref, stats_ref, z_ref, pool_ref):
    p = pl.program_id(0)
    b = pl.program_id(1)

    @pl.when(jnp.logical_and(p == 0, b == 0))
    def _():
        stats_ref[...] = jnp.zeros_like(stats_ref)
        pool_ref[...] = jnp.zeros_like(pool_ref)

    @pl.when(p == 0)
    def _():
        acc = h_ref[...] + a0_ref[...] + a1_ref[...]
        z = jnp.dot(acc, w1_ref[...],
                    preferred_element_type=jnp.float32) + b1_ref[...]
        stats_ref[0:1, :] += jnp.sum(z, axis=0, keepdims=True)
        stats_ref[1:2, :] += jnp.sum(z * z, axis=0, keepdims=True)
        z_ref[pl.ds(b * BLK, BLK), :] = z

    @pl.when(p == 1)
    def _():
        z = z_ref[pl.ds(b * BLK, BLK), :]
        mean = stats_ref[0:1, :] * (1.0 / N)
        var = stats_ref[1:2, :] * (1.0 / N) - mean * mean
        zn = (z - mean) * lax.rsqrt(var + 1e-5) * gb_ref[0:1, :] + gb_ref[1:2, :]
        zr = jnp.maximum(zn, 0.0)
        res = jnp.maximum(
            jnp.dot(zr, w2_ref[...], preferred_element_type=jnp.float32)
            + b2_ref[...], 0.0)
        oh = (batch_ref[...] == lax.broadcasted_iota(
            jnp.int32, (BLK, NGRAPH), 1)).astype(jnp.float32)
        pool_ref[...] += lax.dot_general(
            oh, res, (((0,), (0,)), ((), ())),
            preferred_element_type=jnp.float32)

        @pl.when(b == N // BLK - 1)
        def _():
            g = pool_ref[...]
            t = jnp.maximum(
                jnp.dot(g, l1w_ref[...], preferred_element_type=jnp.float32)
                + l1b_ref[...], 0.0)
            logits = jnp.dot(t, l2w_ref[...],
                             preferred_element_type=jnp.float32) + l2b_ref[...]
            m = jnp.max(logits, axis=1, keepdims=True)
            lse = jnp.log(jnp.sum(jnp.exp(logits - m), axis=1,
                                  keepdims=True)) + m
            out_ref[...] = logits - lse


def _mlp_cls(h, a0, a1, p, batch, params):
    gb = jnp.stack([p["gamma"], p["beta"]])
    return pl.pallas_call(
        _mlp_cls_body,
        grid=(2, N // BLK),
        in_specs=[
            pl.BlockSpec((BLK, F), lambda gp, b: (b * (1 - gp), 0)),
            pl.BlockSpec((BLK, F), lambda gp, b: (b * (1 - gp), 0)),
            pl.BlockSpec((BLK, F), lambda gp, b: (b * (1 - gp), 0)),
            pl.BlockSpec((F, F), lambda gp, b: (0, 0)),
            pl.BlockSpec((1, F), lambda gp, b: (0, 0)),
            pl.BlockSpec((2, F), lambda gp, b: (0, 0)),
            pl.BlockSpec((F, F), lambda gp, b: (0, 0)),
            pl.BlockSpec((1, F), lambda gp, b: (0, 0)),
            pl.BlockSpec((BLK, 1), lambda gp, b: (b * gp, 0)),
            pl.BlockSpec((F, F), lambda gp, b: (0, 0)),
            pl.BlockSpec((1, F), lambda gp, b: (0, 0)),
            pl.BlockSpec((F, NCLS), lambda gp, b: (0, 0)),
            pl.BlockSpec((1, NCLS), lambda gp, b: (0, 0)),
        ],
        out_specs=pl.BlockSpec((NGRAPH, NCLS), lambda gp, b: (0, 0)),
        out_shape=jax.ShapeDtypeStruct((NGRAPH, NCLS), jnp.float32),
        scratch_shapes=[pltpu.VMEM((2, F), jnp.float32),
                        pltpu.VMEM((N, F), jnp.float32),
                        pltpu.VMEM((NGRAPH, F), jnp.float32)],
    )(h, a0, a1, p["w1"], p["b1"][None, :], gb, p["w2"], p["b2"][None, :],
      batch[:, None], params["lin1_w"], params["lin1_b"][None, :],
      params["lin2_w"], params["lin2_b"][None, :])


def _cls_body(h_ref, batch_ref, l1w_ref, l1b_ref, l2w_ref, l2b_ref,
              out_ref, acc_ref):
    b = pl.program_id(0)

    @pl.when(b == 0)
    def _():
        acc_ref[...] = jnp.zeros_like(acc_ref)

    oh = (batch_ref[...] == lax.broadcasted_iota(
        jnp.int32, (BLK, NGRAPH), 1)).astype(jnp.float32)
    acc_ref[...] += lax.dot_general(
        oh, h_ref[...], (((0,), (0,)), ((), ())),
        preferred_element_type=jnp.float32)

    @pl.when(b == N // BLK - 1)
    def _():
        g = acc_ref[...]
        t = jnp.maximum(
            jnp.dot(g, l1w_ref[...], preferred_element_type=jnp.float32)
            + l1b_ref[...], 0.0)
        logits = jnp.dot(t, l2w_ref[...], preferred_element_type=jnp.float32) \
            + l2b_ref[...]
        m = jnp.max(logits, axis=1, keepdims=True)
        lse = jnp.log(jnp.sum(jnp.exp(logits - m), axis=1, keepdims=True)) + m
        out_ref[...] = logits - lse


def _classify(h, batch, params):
    return pl.pallas_call(
        _cls_body,
        grid=(N // BLK,),
        in_specs=[
            pl.BlockSpec((BLK, F), lambda b: (b, 0)),
            pl.BlockSpec((BLK, 1), lambda b: (b, 0)),
            pl.BlockSpec((F, F), lambda b: (0, 0)),
            pl.BlockSpec((1, F), lambda b: (0, 0)),
            pl.BlockSpec((F, NCLS), lambda b: (0, 0)),
            pl.BlockSpec((1, NCLS), lambda b: (0, 0)),
        ],
        out_specs=pl.BlockSpec((NGRAPH, NCLS), lambda b: (0, 0)),
        out_shape=jax.ShapeDtypeStruct((NGRAPH, NCLS), jnp.float32),
        scratch_shapes=[pltpu.VMEM((NGRAPH, F), jnp.float32)],
    )(h, batch[:, None], params["lin1_w"], params["lin1_b"][None, :],
      params["lin2_w"], params["lin2_b"][None, :])


def kernel(x, edge_index, edge_weight, batch, params):
    src = edge_index[0].reshape(NW, NCHUNK, 1, CHUNK)
    dst = edge_index[1].reshape(NW, NCHUNK, 1, CHUNK)
    ew = edge_weight.reshape(NW, NCHUNK, 1, CHUNK)
    h = x
    for name in ("conv1", "conv2"):
        a0, a1 = _aggregate(h, src, dst, ew)
        h = _mlp(h, a0, a1, params[name])
    a0, a1 = _aggregate(h, src, dst, ew)
    return _mlp_cls(h, a0, a1, params["conv3"], batch, params)


# speedup vs baseline: 1.0277x; 1.0017x over previous
import jax
import jax.numpy as jnp
from jax import lax
from jax.experimental import pallas as pl
from jax.experimental.pallas import tpu as pltpu
from jax.experimental.pallas import tpu_sc as plsc

N = 10000
E = 320000
F = 128
FH = F // 2
NGRAPH = 128
NCLS = 10

NC = 2
NS = 16
NW = NC * NS
EPW = E // NW
CHUNK = 80
NCHUNK = EPW // CHUNK
RPT = 624
ZBLK = 48
REM = N - NS * RPT

_mesh = plsc.VectorSubcoreMesh(core_axis_name="c", subcore_axis_name="s")


def _lane_bcast(vec, lane):
    idx = jnp.full((16, 1), lane, jnp.int32)
    return lax.gather(
        vec, idx,
        lax.GatherDimensionNumbers(
            offset_dims=(), collapsed_slice_dims=(0,), start_index_map=(0,)),
        slice_sizes=(1,),
        mode=lax.GatherScatterMode.PROMISE_IN_BOUNDS)


def _aggregate_body(h_hbm, src_hbm, dst_hbm, ew_hbm, out0_hbm, out1_hbm,
                    srcv0, srcv1, dstv0, dstv1, ewv0, ewv1,
                    rowsv0, rowsv1, obuf, accum,
                    sem_i0, sem_i1, sem_r0, sem_r1):
    c = lax.axis_index("c")
    s = lax.axis_index("s")
    w = s * NC + c

    def zrow(r, carry):
        for k in range(F // 16):
            obuf[r, pl.ds(k * 16, 16)] = jnp.zeros((16,), jnp.float32)
        return carry
    lax.fori_loop(0, ZBLK, zrow, 0)
    for k in range(RPT // ZBLK):
        pltpu.sync_copy(obuf, accum.at[pl.ds(s * RPT + k * ZBLK, ZBLK)])

    @pl.when(s == NS - 1)
    def _():
        pltpu.sync_copy(obuf.at[pl.ds(0, REM)], accum.at[pl.ds(NS * RPT, REM)])
    plsc.subcore_barrier()

    srcv = (srcv0, srcv1)
    dstv = (dstv0, dstv1)
    ewv = (ewv0, ewv1)
    rowsv = (rowsv0, rowsv1)
    sem_i = (sem_i0, sem_i1)
    sem_r = (sem_r0, sem_r1)

    def stage_idx(i, p):
        pltpu.async_copy(src_hbm.at[w, i], srcv[p], sem_i[p])
        pltpu.async_copy(dst_hbm.at[w, i], dstv[p], sem_i[p])
        pltpu.async_copy(ew_hbm.at[w, i], ewv[p], sem_i[p])

    def wait_idx(i, p):
        pltpu.make_async_copy(src_hbm.at[w, i], srcv[p], sem_i[p]).wait()
        pltpu.make_async_copy(dst_hbm.at[w, i], dstv[p], sem_i[p]).wait()
        pltpu.make_async_copy(ew_hbm.at[w, i], ewv[p], sem_i[p]).wait()

    def body(i, p):
        pltpu.make_async_copy(h_hbm.at[srcv[p].at[0]], rowsv[p],
                              sem_r[p]).wait()

        @pl.when(i + 1 < NCHUNK)
        def _():
            wait_idx(i + 1, 1 - p)
            pltpu.async_copy(h_hbm.at[srcv[1 - p].at[0]], rowsv[1 - p],
                             sem_r[1 - p])

        for j in range(CHUNK // 16):
            ew16 = ewv[p][0, pl.ds(j * 16, 16)]
            for l in range(16):
                w16 = _lane_bcast(ew16, l)
                e = j * 16 + l
                for k in range(F // 16):
                    sl = pl.ds(k * 16, 16)
                    rowsv[p][e, sl] = rowsv[p][e, sl] * w16

        pltpu.sync_copy(rowsv[p], accum.at[dstv[p].at[0]], add=True)

        @pl.when(i + 2 < NCHUNK)
        def _():
            stage_idx(i + 2, p)

    stage_idx(0, 0)
    stage_idx(1, 1)
    wait_idx(0, 0)
    pltpu.async_copy(h_hbm.at[srcv[0].at[0]], rowsv[0], sem_r[0])

    def pair(g, carry):
        body(g, 0)
        body(g + 1, 1)
        return carry
    lax.fori_loop(0, (NCHUNK - 1) // 2, lambda g, cc: pair(2 * g, cc), 0)
    body(NCHUNK - 1, 0)
    plsc.subcore_barrier()

    def writeout(out_ref):
        for k in range(RPT // ZBLK):
            base = s * RPT + k * ZBLK
            pltpu.sync_copy(accum.at[pl.ds(base, ZBLK)], obuf)
            pltpu.sync_copy(obuf, out_ref.at[pl.ds(base, ZBLK)])

        @pl.when(s == NS - 1)
        def _():
            pltpu.sync_copy(accum.at[pl.ds(NS * RPT, REM)],
                            obuf.at[pl.ds(0, REM)])
            pltpu.sync_copy(obuf.at[pl.ds(0, REM)],
                            out_ref.at[pl.ds(NS * RPT, REM)])

    @pl.when(c == 0)
    def _():
        writeout(out0_hbm)

    @pl.when(c == 1)
    def _():
        writeout(out1_hbm)


_aggregate = pl.kernel(
    _aggregate_body,
    out_type=(jax.ShapeDtypeStruct((N, F), jnp.float32),
              jax.ShapeDtypeStruct((N, F), jnp.float32)),
    mesh=_mesh,
    scratch_types=[
        pltpu.VMEM((1, CHUNK), jnp.int32),
        pltpu.VMEM((1, CHUNK), jnp.int32),
        pltpu.VMEM((1, CHUNK), jnp.int32),
        pltpu.VMEM((1, CHUNK), jnp.int32),
        pltpu.VMEM((1, CHUNK), jnp.float32),
        pltpu.VMEM((1, CHUNK), jnp.float32),
        pltpu.VMEM((CHUNK, F), jnp.float32),
        pltpu.VMEM((CHUNK, F), jnp.float32),
        pltpu.VMEM((ZBLK, F), jnp.float32),
        pltpu.VMEM_SHARED((N, F), jnp.float32),
        pltpu.SemaphoreType.DMA,
        pltpu.SemaphoreType.DMA,
        pltpu.SemaphoreType.DMA,
        pltpu.SemaphoreType.DMA,
    ],
)


BLK = 1000


def _mlp_body(h_ref, a0_ref, a1_ref, w1_ref, b1_ref, gb_ref,
              w2_ref, b2_ref, out_ref, stats_ref, z_ref):
    p = pl.program_id(0)
    b = pl.program_id(1)

    @pl.when(jnp.logical_and(p == 0, b == 0))
    def _():
        stats_ref[...] = jnp.zeros_like(stats_ref)

    @pl.when(p == 0)
    def _():
        acc = h_ref[...] + a0_ref[...] + a1_ref[...]
        z = jnp.dot(acc, w1_ref[...],
                    preferred_element_type=jnp.float32) + b1_ref[...]
        stats_ref[0:1, :] += jnp.sum(z, axis=0, keepdims=True)
        stats_ref[1:2, :] += jnp.sum(z * z, axis=0, keepdims=True)
        z_ref[pl.ds(b * BLK, BLK), :] = z

    @pl.when(p == 1)
    def _():
        z = z_ref[pl.ds(b * BLK, BLK), :]
        mean = stats_ref[0:1, :] * (1.0 / N)
        var = stats_ref[1:2, :] * (1.0 / N) - mean * mean
        zn = (z - mean) * lax.rsqrt(var + 1e-5) * gb_ref[0:1, :] + gb_ref[1:2, :]
        zr = jnp.maximum(zn, 0.0)
        out_ref[...] = jnp.maximum(
            jnp.dot(zr, w2_ref[...], preferred_element_type=jnp.float32)
            + b2_ref[...], 0.0)


def _mlp(h, a0, a1, p):
    gb = jnp.stack([p["gamma"], p["beta"]])
    return pl.pallas_call(
        _mlp_body,
        grid=(2, N // BLK),
        in_specs=[
            pl.BlockSpec((BLK, F), lambda gp, b: (b * (1 - gp), 0)),
            pl.BlockSpec((BLK, F), lambda gp, b: (b * (1 - gp), 0)),
            pl.BlockSpec((BLK, F), lambda gp, b: (b * (1 - gp), 0)),
            pl.BlockSpec((F, F), lambda gp, b: (0, 0)),
            pl.BlockSpec((1, F), lambda gp, b: (0, 0)),
            pl.BlockSpec((2, F), lambda gp, b: (0, 0)),
            pl.BlockSpec((F, F), lambda gp, b: (0, 0)),
            pl.BlockSpec((1, F), lambda gp, b: (0, 0)),
        ],
        out_specs=pl.BlockSpec((BLK, F), lambda gp, b: (b * gp, 0)),
        out_shape=jax.ShapeDtypeStruct((N, F), jnp.float32),
        scratch_shapes=[pltpu.VMEM((2, F), jnp.float32),
                        pltpu.VMEM((N, F), jnp.float32)],
    )(h, a0, a1, p["w1"], p["b1"][None, :], gb, p["w2"], p["b2"][None, :])


def _mlp_cls_body(h_ref, a0_ref, a1_ref, w1_ref, b1_ref, gb_ref,
                  w2_ref, b2_ref, batch_ref, l1w_ref, l1b_ref,
                  l2w_ref, l2b_ref, out_ref, stats_ref, z_ref, pool_ref):
    p = pl.program_id(0)
    b = pl.program_id(1)

    @pl.when(jnp.logical_and(p == 0, b == 0))
    def _():
        stats_ref[...] = jnp.zeros_like(stats_ref)
        pool_ref[...] = jnp.zeros_like(pool_ref)

    @pl.when(p == 0)
    def _():
        acc = h_ref[...] + a0_ref[...] + a1_ref[...]
        z = jnp.dot(acc, w1_ref[...],
                    preferred_element_type=jnp.float32) + b1_ref[...]
        stats_ref[0:1, :] += jnp.sum(z, axis=0, keepdims=True)
        stats_ref[1:2, :] += jnp.sum(z * z, axis=0, keepdims=True)
        z_ref[pl.ds(b * BLK, BLK), :] = z

    @pl.when(p == 1)
    def _():
        z = z_ref[pl.ds(b * BLK, BLK), :]
        mean = stats_ref[0:1, :] * (1.0 / N)
        var = stats_ref[1:2, :] * (1.0 / N) - mean * mean
        zn = (z - mean) * lax.rsqrt(var + 1e-5) * gb_ref[0:1, :] + gb_ref[1:2, :]
        zr = jnp.maximum(zn, 0.0)
        res = jnp.maximum(
            jnp.dot(zr, w2_ref[...], preferred_element_type=jnp.float32)
            + b2_ref[...], 0.0)
        oh = (batch_ref[...] == lax.broadcasted_iota(
            jnp.int32, (BLK, NGRAPH), 1)).astype(jnp.float32)
        pool_ref[...] += lax.dot_general(
            oh, res, (((0,), (0,)), ((), ())),
            preferred_element_type=jnp.float32)

        @pl.when(b == N // BLK - 1)
        def _():
            g = pool_ref[...]
            t = jnp.maximum(
                jnp.dot(g, l1w_ref[...], preferred_element_type=jnp.float32)
                + l1b_ref[...], 0.0)
            logits = jnp.dot(t, l2w_ref[...],
                             preferred_element_type=jnp.float32) + l2b_ref[...]
            m = jnp.max(logits, axis=1, keepdims=True)
            lse = jnp.log(jnp.sum(jnp.exp(logits - m), axis=1,
                                  keepdims=True)) + m
            out_ref[...] = logits - lse


def _mlp_cls(h, a0, a1, p, batch, params):
    gb = jnp.stack([p["gamma"], p["beta"]])
    return pl.pallas_call(
        _mlp_cls_body,
        grid=(2, N // BLK),
        in_specs=[
            pl.BlockSpec((BLK, F), lambda gp, b: (b * (1 - gp), 0)),
            pl.BlockSpec((BLK, F), lambda gp, b: (b * (1 - gp), 0)),
            pl.BlockSpec((BLK, F), lambda gp, b: (b * (1 - gp), 0)),
            pl.BlockSpec((F, F), lambda gp, b: (0, 0)),
            pl.BlockSpec((1, F), lambda gp, b: (0, 0)),
            pl.BlockSpec((2, F), lambda gp, b: (0, 0)),
            pl.BlockSpec((F, F), lambda gp, b: (0, 0)),
            pl.BlockSpec((1, F), lambda gp, b: (0, 0)),
            pl.BlockSpec((BLK, 1), lambda gp, b: (b * gp, 0)),
            pl.BlockSpec((F, F), lambda gp, b: (0, 0)),
            pl.BlockSpec((1, F), lambda gp, b: (0, 0)),
            pl.BlockSpec((F, NCLS), lambda gp, b: (0, 0)),
            pl.BlockSpec((1, NCLS), lambda gp, b: (0, 0)),
        ],
        out_specs=pl.BlockSpec((NGRAPH, NCLS), lambda gp, b: (0, 0)),
        out_shape=jax.ShapeDtypeStruct((NGRAPH, NCLS), jnp.float32),
        scratch_shapes=[pltpu.VMEM((2, F), jnp.float32),
                        pltpu.VMEM((N, F), jnp.float32),
                        pltpu.VMEM((NGRAPH, F), jnp.float32)],
    )(h, a0, a1, p["w1"], p["b1"][None, :], gb, p["w2"], p["b2"][None, :],
      batch[:, None], params["lin1_w"], params["lin1_b"][None, :],
      params["lin2_w"], params["lin2_b"][None, :])


def kernel(x, edge_index, edge_weight, batch, params):
    src = edge_index[0].reshape(NW, NCHUNK, 1, CHUNK)
    dst = edge_index[1].reshape(NW, NCHUNK, 1, CHUNK)
    ew = edge_weight.reshape(NW, NCHUNK, 1, CHUNK)
    h = x
    for name in ("conv1", "conv2"):
        a0, a1 = _aggregate(h, src, dst, ew)
        h = _mlp(h, a0, a1, params[name])
    a0, a1 = _aggregate(h, src, dst, ew)
    return _mlp_cls(h, a0, a1, params["conv3"], batch, params)
